# baseline (speedup 1.0000x reference)
import jax
import jax.numpy as jnp
from jax import lax
from jax.experimental import pallas as pl
from jax.experimental.pallas import tpu as pltpu
from jax.experimental.pallas import tpu_sc as plsc

_B = 16384
_H = 200
_NUM_WORKERS = 32
_ROWS_W = _B // _NUM_WORKERS
_CROWS = 64
_CHUNK = _CROWS * _H
_N_CHUNKS = _ROWS_W // _CROWS
_LANES = 16

_OFFS = list(range(0, _H - _LANES + 1, _LANES)) + [_H - _LANES]


def _flatten_rows(src2d, dst_flat):
    def _body(r, carry):
        row = src2d.at[r]
        for o in _OFFS:
            dst_flat[pl.ds(r * _H + o, _LANES)] = row[pl.ds(o, _LANES)]
        return carry

    lax.fori_loop(0, _CROWS, _body, 0)


def _subtract_to_2d(src_flat, dst2d):
    def _body(r, carry):
        row = dst2d.at[r]
        for o in _OFFS:
            v = src_flat[pl.ds(r * _H + o, _LANES)]
            row[pl.ds(o, _LANES)] = v - v
        return carry

    lax.fori_loop(0, _CROWS, _body, 0)


def _witness_body(ids_hbm, tab_hbm, out_hbm,
                  idx2d_v0, idx2d_v1, idxf_v0, idxf_v1,
                  rowsf_v0, rowsf_v1, rows2d_v0, rows2d_v1,
                  gsem0, gsem1, osem0, osem1):
    wid = lax.axis_index("s") * 2 + lax.axis_index("c")
    rbase = wid * _ROWS_W
    idx2d_v = (idx2d_v0, idx2d_v1)
    idxf_v = (idxf_v0, idxf_v1)
    rowsf_v = (rowsf_v0, rowsf_v1)
    rows2d_v = (rows2d_v0, rows2d_v1)
    gsem = (gsem0, gsem1)
    osem = (osem0, osem1)

    def _stage_and_fire(g, b):
        pltpu.sync_copy(ids_hbm.at[pl.ds(rbase + g * _CROWS, _CROWS)],
                        idx2d_v[b])
        _flatten_rows(idx2d_v[b], idxf_v[b])
        return pltpu.async_copy(tab_hbm.at[idxf_v[b]], rowsf_v[b], gsem[b])

    gat = {0: _stage_and_fire(0, 0)}
    out_cp = {}

    for g in range(_N_CHUNKS):
        b = g % 2
        if g + 1 < _N_CHUNKS:
            gat[g + 1] = _stage_and_fire(g + 1, 1 - b)
        gat[g].wait()
        if g - 2 >= 0:
            out_cp[g - 2].wait()
        _subtract_to_2d(rowsf_v[b], rows2d_v[b])
        out_cp[g] = pltpu.async_copy(
            rows2d_v[b], out_hbm.at[pl.ds(rbase + g * _CROWS, _CROWS)],
            osem[b])
    out_cp[_N_CHUNKS - 2].wait()
    out_cp[_N_CHUNKS - 1].wait()


def kernel(input_ids, witness_ids, witness_weight):
    del input_ids
    tab = witness_weight.reshape(-1)
    mesh = plsc.VectorSubcoreMesh(core_axis_name="c", subcore_axis_name="s")
    out = pl.kernel(
        _witness_body,
        out_type=jax.ShapeDtypeStruct((_B, _H), jnp.float32),
        mesh=mesh,
        scratch_types=[
            pltpu.VMEM((_CROWS, _H), jnp.int32),
            pltpu.VMEM((_CROWS, _H), jnp.int32),
            pltpu.VMEM((_CHUNK,), jnp.int32),
            pltpu.VMEM((_CHUNK,), jnp.int32),
            pltpu.VMEM((_CHUNK,), jnp.float32),
            pltpu.VMEM((_CHUNK,), jnp.float32),
            pltpu.VMEM((_CROWS, _H), jnp.float32),
            pltpu.VMEM((_CROWS, _H), jnp.float32),
            pltpu.SemaphoreType.DMA,
            pltpu.SemaphoreType.DMA,
            pltpu.SemaphoreType.DMA,
            pltpu.SemaphoreType.DMA,
        ],
    )(witness_ids, tab)
    return out.reshape(_B, _H, 1)
